# Initial kernel scaffold; baseline (speedup 1.0000x reference)
import jax
import jax.numpy as jnp
from jax.experimental import pallas as pl


def kernel(x_paper, x_author, ei_writes, ei_cites, W_pre_paper, b_pre_paper, W_k_paper, b_k_paper, W_q_paper, b_q_paper, W_v_paper, b_v_paper, W_a_paper, b_a_paper, skip_paper, W_pre_author, b_pre_author, W_k_author, b_k_author, W_q_author, b_q_author, W_v_author, b_v_author, W_a_author, b_a_author, skip_author, a_rel_writes, m_rel_writes, p_rel_writes, a_rel_cites, m_rel_cites, p_rel_cites, W_out, b_out):
    raise NotImplementedError("write your pallas kernel here")


# speedup vs baseline: 1.9137x; 1.9137x over previous
import functools

import jax
import jax.numpy as jnp
from jax.experimental import pallas as pl

H = 8
D = 128
DH = D // H
OUT = 16


def _block_diag(rel):
    eye = jnp.eye(H, dtype=rel.dtype)
    return jnp.einsum('hg,hij->higj', eye, rel).reshape(D, D)


def _proj_body(x_ref, w_ref, b_ref, o_ref):
    o_ref[...] = jnp.dot(x_ref[...], w_ref[...],
                         preferred_element_type=jnp.float32) + b_ref[...]


def _proj(x, W, b, bn=1000):
    n, d = x.shape
    c = W.shape[1]
    grid = (n // bn,)
    return pl.pallas_call(
        _proj_body,
        grid=grid,
        in_specs=[
            pl.BlockSpec((bn, d), lambda i: (i, 0)),
            pl.BlockSpec((d, c), lambda i: (0, 0)),
            pl.BlockSpec((1, c), lambda i: (0, 0)),
        ],
        out_specs=pl.BlockSpec((bn, c), lambda i: (i, 0)),
        out_shape=jax.ShapeDtypeStruct((n, c), jnp.float32),
    )(x, W, b.reshape(1, c))


def _post_body(agg_ref, xo_ref, wao_ref, c0_ref, o_ref):
    g = jax.nn.gelu(agg_ref[...])
    o_ref[...] = (jnp.dot(g, wao_ref[...], preferred_element_type=jnp.float32)
                  + xo_ref[...] + c0_ref[...])


def _post(aggr, xo, W_ao, c0, bn=1000):
    n = aggr.shape[0]
    grid = (n // bn,)
    return pl.pallas_call(
        _post_body,
        grid=grid,
        in_specs=[
            pl.BlockSpec((bn, D), lambda i: (i, 0)),
            pl.BlockSpec((bn, OUT), lambda i: (i, 0)),
            pl.BlockSpec((D, OUT), lambda i: (0, 0)),
            pl.BlockSpec((1, OUT), lambda i: (0, 0)),
        ],
        out_specs=pl.BlockSpec((bn, OUT), lambda i: (i, 0)),
        out_shape=jax.ShapeDtypeStruct((n, OUT), jnp.float32),
    )(aggr, xo, W_ao, c0.reshape(1, OUT))


def kernel(x_paper, x_author, ei_writes, ei_cites, W_pre_paper, b_pre_paper,
           W_k_paper, b_k_paper, W_q_paper, b_q_paper, W_v_paper, b_v_paper,
           W_a_paper, b_a_paper, skip_paper, W_pre_author, b_pre_author,
           W_k_author, b_k_author, W_q_author, b_q_author, W_v_author,
           b_v_author, W_a_author, b_a_author, skip_author, a_rel_writes,
           m_rel_writes, p_rel_writes, a_rel_cites, m_rel_cites, p_rel_cites,
           W_out, b_out):
    n_p = x_paper.shape[0]
    n_a = x_author.shape[0]

    inv_sqrt = 1.0 / jnp.sqrt(jnp.float32(DH))
    A_c = _block_diag(a_rel_cites) * jnp.repeat(p_rel_cites * inv_sqrt, DH)[None, :]
    M_c = _block_diag(m_rel_cites)
    A_w = _block_diag(a_rel_writes) * jnp.repeat(p_rel_writes * inv_sqrt, DH)[None, :]
    M_w = _block_diag(m_rel_writes)

    beta = jax.nn.sigmoid(skip_paper)

    Wq_p = W_pre_paper @ W_q_paper
    bq_p = b_pre_paper @ W_q_paper + b_q_paper
    Wk_p = W_pre_paper @ (W_k_paper @ A_c)
    bk_p = (b_pre_paper @ W_k_paper + b_k_paper) @ A_c
    Wv_p = W_pre_paper @ (W_v_paper @ M_c)
    bv_p = (b_pre_paper @ W_v_paper + b_v_paper) @ M_c
    Wk_a = W_pre_author @ (W_k_author @ A_w)
    bk_a = (b_pre_author @ W_k_author + b_k_author) @ A_w
    Wv_a = W_pre_author @ (W_v_author @ M_w)
    bv_a = (b_pre_author @ W_v_author + b_v_author) @ M_w

    W_ao = (W_a_paper @ W_out) * beta
    W_xo = (W_pre_paper @ W_out) * (1.0 - beta)
    c0 = (beta * (b_a_paper @ W_out) + (1.0 - beta) * (b_pre_paper @ W_out)
          + b_out)

    Wxo_pad = jnp.zeros((D, D), jnp.float32).at[:, :OUT].set(W_xo)
    Wcat_p = jnp.concatenate([Wq_p, Wk_p, Wv_p, Wxo_pad], axis=1)
    bcat_p = jnp.concatenate([bq_p, bk_p, bv_p, jnp.zeros((D,), jnp.float32)])
    proj_p = _proj(x_paper, Wcat_p, bcat_p)
    q_p = proj_p[:, 0:D]
    k_p = proj_p[:, D:2 * D]
    v_p = proj_p[:, 2 * D:3 * D]
    xo = proj_p[:, 3 * D:3 * D + OUT]

    Wcat_a = jnp.concatenate([Wk_a, Wv_a], axis=1)
    bcat_a = jnp.concatenate([bk_a, bv_a])
    proj_a = _proj(x_author, Wcat_a, bcat_a)
    k_a = proj_a[:, 0:D]
    v_a = proj_a[:, D:2 * D]

    def edge_phase(q_dst, k_src, v_src, s_idx, d_idx):
        qh = jnp.take(q_dst, d_idx, axis=0).reshape(-1, H, DH)
        kh = jnp.take(k_src, s_idx, axis=0).reshape(-1, H, DH)
        w = jnp.exp((qh * kh).sum(-1))
        vh = jnp.take(v_src, s_idx, axis=0).reshape(-1, H, DH)
        msg = vh * w[:, :, None]
        accum = jax.ops.segment_sum(msg, d_idx, num_segments=n_p)
        denom = jax.ops.segment_sum(w, d_idx, num_segments=n_p)
        return accum / (denom[:, :, None] + 1e-16)

    aggr = (edge_phase(q_p, k_a, v_a, ei_writes[0], ei_writes[1])
            + edge_phase(q_p, k_p, v_p, ei_cites[0], ei_cites[1]))

    return _post(aggr.reshape(n_p, D), xo, W_ao, c0)


# speedup vs baseline: 25.6194x; 13.3871x over previous
import jax
import jax.numpy as jnp
from jax import lax
from jax.experimental import pallas as pl
from jax.experimental.pallas import tpu as pltpu
from jax.experimental.pallas import tpu_sc as plsc

H = 8
D = 128
DH = D // H
OUT = 16

N_TEC = 16
N_SC = 2
CHUNK = 128


def _block_diag(rel):
    eye = jnp.eye(H, dtype=rel.dtype)
    return jnp.einsum('hg,hij->higj', eye, rel).reshape(D, D)


def _proj_body(x_ref, w_ref, b_ref, o_ref):
    o_ref[...] = jnp.dot(x_ref[...], w_ref[...],
                         preferred_element_type=jnp.float32) + b_ref[...]


def _proj(x, W, b, bn=1000):
    n, d = x.shape
    c = W.shape[1]
    return pl.pallas_call(
        _proj_body,
        grid=(n // bn,),
        in_specs=[
            pl.BlockSpec((bn, d), lambda i: (i, 0)),
            pl.BlockSpec((d, c), lambda i: (0, 0)),
            pl.BlockSpec((1, c), lambda i: (0, 0)),
        ],
        out_specs=pl.BlockSpec((bn, c), lambda i: (i, 0)),
        out_shape=jax.ShapeDtypeStruct((n, c), jnp.float32),
    )(x, W, b.reshape(1, c))


def _edge_phase_sc(q_tab, kp_tab, vp_tab, ka_tab, va_tab,
                   s_w, d_w, s_c, d_c, n_p, n_a, e_pad):
    per_tec = e_pad // N_TEC
    n_chunks = per_tec // CHUNK
    DRN = 112
    acc_rows = ((n_p + N_TEC * DRN - 1) // (N_TEC * DRN)) * (N_TEC * DRN)
    rows_per_tec = acc_rows // N_TEC
    n_drain = rows_per_tec // DRN

    mesh = plsc.VectorSubcoreMesh(core_axis_name="c", subcore_axis_name="s")

    def body(q_hbm, kp_hbm, vp_hbm, ka_hbm, va_hbm,
             sw_hbm, dw_hbm, sc_hbm, dc_hbm, out_hbm,
             dbuf, sbuf, qidx, sidx, qb, kb, vb, mb, zbuf, wbuf, accum,
             sem, sem2, sem3):
        cid = lax.axis_index("c")
        sid = lax.axis_index("s")
        iota16 = lax.iota(jnp.int32, 16)
        e0f = jnp.where(iota16 == 0, 1.0, 0.0)

        z16 = jnp.zeros((16,), jnp.float32)

        def zb(i, _):
            zbuf[i, pl.ds(0, 16)] = z16
            zbuf[i, pl.ds(16, 16)] = z16
            return _
        lax.fori_loop(0, zbuf.shape[0], zb, None)

        for i in range(8):
            t = i // 4
            h = 2 * (i % 4) + cid
            u = t * 8 + h
            if t == 0:
                s_hbm, d_hbm, k_hbm, v_hbm, n_src = (
                    sw_hbm, dw_hbm, ka_hbm, va_hbm, n_a)
            else:
                s_hbm, d_hbm, k_hbm, v_hbm, n_src = (
                    sc_hbm, dc_hbm, kp_hbm, vp_hbm, n_p)

            def zcp(z, _):
                r0 = sid * rows_per_tec + z * DRN
                pltpu.sync_copy(zbuf, accum.at[pl.ds(r0, DRN)])
                return _
            lax.fori_loop(0, n_drain, zcp, None)
            plsc.subcore_barrier()

            qoff = h * n_p
            koff = h * n_src

            def chunk_body(ch, _):
                off = sid * per_tec + ch * CHUNK
                pltpu.sync_copy(d_hbm.at[pl.ds(off, CHUNK)], dbuf.at[0])
                pltpu.sync_copy(s_hbm.at[pl.ds(off, CHUNK)], sbuf)
                for j in range(CHUNK // 16):
                    dv = dbuf[0, pl.ds(j * 16, 16)]
                    qidx[pl.ds(j * 16, 16)] = (
                        jnp.minimum(dv, n_p - 1) + qoff)
                    sidx[pl.ds(j * 16, 16)] = sbuf[pl.ds(j * 16, 16)] + koff
                c1 = pltpu.async_copy(q_hbm.at[qidx], qb, sem)
                c2 = pltpu.async_copy(k_hbm.at[sidx], kb, sem2)
                c3 = pltpu.async_copy(v_hbm.at[sidx], vb, sem3)
                c1.wait(); c2.wait(); c3.wait()

                def group(g, carry):
                    e0 = g * 16
                    rows = e0 + iota16
                    acc = jnp.zeros((16,), jnp.float32)
                    for dh in range(DH):
                        col = jnp.full((16,), dh, jnp.int32)
                        acc = acc + (plsc.load_gather(qb, [rows, col])
                                     * plsc.load_gather(kb, [rows, col]))
                    wv = jnp.exp(acc)
                    for e in range(16):
                        ws = wv[e]
                        mb[e0 + e, pl.ds(0, 16)] = vb[e0 + e] * ws
                        mb[e0 + e, pl.ds(16, 16)] = ws * e0f
                    return carry
                lax.fori_loop(0, CHUNK // 16, group, None)
                pltpu.sync_copy(mb, accum.at[dbuf.at[0]], add=True)
                return _
            lax.fori_loop(0, n_chunks, chunk_body, None)
            plsc.subcore_barrier()

            def drain(dr, _):
                r0 = sid * rows_per_tec + dr * DRN
                pltpu.sync_copy(accum.at[pl.ds(r0, DRN)],
                                mb.at[pl.ds(0, DRN)])
                pltpu.sync_copy(mb.at[pl.ds(0, DRN)],
                                out_hbm.at[u, pl.ds(r0, DRN)])
                return _
            lax.fori_loop(0, n_drain, drain, None)
            plsc.subcore_barrier()

    fn = pl.kernel(
        body,
        out_type=jax.ShapeDtypeStruct((16, acc_rows, 32), jnp.float32),
        mesh=mesh,
        compiler_params=pltpu.CompilerParams(needs_layout_passes=False,
                                             use_tc_tiling_on_sc=False),
        scratch_types=[
            pltpu.VMEM((1, CHUNK), jnp.int32),
            pltpu.VMEM((CHUNK,), jnp.int32),
            pltpu.VMEM((CHUNK,), jnp.int32),
            pltpu.VMEM((CHUNK,), jnp.int32),
            pltpu.VMEM((CHUNK, 16), jnp.float32),
            pltpu.VMEM((CHUNK, 16), jnp.float32),
            pltpu.VMEM((CHUNK, 16), jnp.float32),
            pltpu.VMEM((CHUNK, 32), jnp.float32),
            pltpu.VMEM((DRN, 32), jnp.float32),
            pltpu.VMEM((16,), jnp.float32),
            pltpu.VMEM_SHARED((acc_rows, 32), jnp.float32),
            pltpu.SemaphoreType.DMA,
            pltpu.SemaphoreType.DMA,
            pltpu.SemaphoreType.DMA,
        ],
    )
    return fn(q_tab, kp_tab, vp_tab, ka_tab, va_tab, s_w, d_w, s_c, d_c)


def _post_body(sc_ref, xo_ref, wao_ref, c0_ref, o_ref):
    o = xo_ref[...] + c0_ref[...]
    for h in range(H):
        aggr_h = (sc_ref[h, :, 0:16] / (sc_ref[h, :, 16:17] + 1e-16)
                  + sc_ref[8 + h, :, 0:16] / (sc_ref[8 + h, :, 16:17] + 1e-16))
        g = jax.nn.gelu(aggr_h)
        o = o + jnp.dot(g, wao_ref[pl.ds(h * DH, DH), :],
                        preferred_element_type=jnp.float32)
    o_ref[...] = o


def _post(sc_out, xo, W_ao, c0, bn=1000):
    n = xo.shape[0]
    return pl.pallas_call(
        _post_body,
        grid=(n // bn,),
        in_specs=[
            pl.BlockSpec((16, bn, 32), lambda i: (0, i, 0)),
            pl.BlockSpec((bn, OUT), lambda i: (i, 0)),
            pl.BlockSpec((D, OUT), lambda i: (0, 0)),
            pl.BlockSpec((1, OUT), lambda i: (0, 0)),
        ],
        out_specs=pl.BlockSpec((bn, OUT), lambda i: (i, 0)),
        out_shape=jax.ShapeDtypeStruct((n, OUT), jnp.float32),
    )(sc_out, xo, W_ao, c0.reshape(1, OUT))


def kernel(x_paper, x_author, ei_writes, ei_cites, W_pre_paper, b_pre_paper,
           W_k_paper, b_k_paper, W_q_paper, b_q_paper, W_v_paper, b_v_paper,
           W_a_paper, b_a_paper, skip_paper, W_pre_author, b_pre_author,
           W_k_author, b_k_author, W_q_author, b_q_author, W_v_author,
           b_v_author, W_a_author, b_a_author, skip_author, a_rel_writes,
           m_rel_writes, p_rel_writes, a_rel_cites, m_rel_cites, p_rel_cites,
           W_out, b_out):
    n_p = x_paper.shape[0]
    n_a = x_author.shape[0]
    e_w = ei_writes.shape[1]
    e_c = ei_cites.shape[1]

    inv_sqrt = 1.0 / jnp.sqrt(jnp.float32(DH))
    A_c = _block_diag(a_rel_cites) * jnp.repeat(p_rel_cites * inv_sqrt, DH)[None, :]
    M_c = _block_diag(m_rel_cites)
    A_w = _block_diag(a_rel_writes) * jnp.repeat(p_rel_writes * inv_sqrt, DH)[None, :]
    M_w = _block_diag(m_rel_writes)

    beta = jax.nn.sigmoid(skip_paper)

    Wq_p = W_pre_paper @ W_q_paper
    bq_p = b_pre_paper @ W_q_paper + b_q_paper
    Wk_p = W_pre_paper @ (W_k_paper @ A_c)
    bk_p = (b_pre_paper @ W_k_paper + b_k_paper) @ A_c
    Wv_p = W_pre_paper @ (W_v_paper @ M_c)
    bv_p = (b_pre_paper @ W_v_paper + b_v_paper) @ M_c
    Wk_a = W_pre_author @ (W_k_author @ A_w)
    bk_a = (b_pre_author @ W_k_author + b_k_author) @ A_w
    Wv_a = W_pre_author @ (W_v_author @ M_w)
    bv_a = (b_pre_author @ W_v_author + b_v_author) @ M_w

    W_ao = (W_a_paper @ W_out) * beta
    W_xo = (W_pre_paper @ W_out) * (1.0 - beta)
    c0 = (beta * (b_a_paper @ W_out) + (1.0 - beta) * (b_pre_paper @ W_out)
          + b_out)

    Wxo_pad = jnp.zeros((D, D), jnp.float32).at[:, :OUT].set(W_xo)
    Wcat_p = jnp.concatenate([Wq_p, Wk_p, Wv_p, Wxo_pad], axis=1)
    bcat_p = jnp.concatenate([bq_p, bk_p, bv_p, jnp.zeros((D,), jnp.float32)])
    proj_p = _proj(x_paper, Wcat_p, bcat_p)
    xo = proj_p[:, 3 * D:3 * D + OUT]

    Wcat_a = jnp.concatenate([Wk_a, Wv_a], axis=1)
    bcat_a = jnp.concatenate([bk_a, bv_a])
    proj_a = _proj(x_author, Wcat_a, bcat_a)

    def headmajor(m):
        n = m.shape[0]
        return m.reshape(n, H, DH).transpose(1, 0, 2).reshape(H * n, DH)

    q_tab = headmajor(proj_p[:, 0:D])
    kp_tab = headmajor(proj_p[:, D:2 * D])
    vp_tab = headmajor(proj_p[:, 2 * D:3 * D])
    ka_tab = headmajor(proj_a[:, 0:D])
    va_tab = headmajor(proj_a[:, D:2 * D])

    grain = N_TEC * CHUNK
    e_pad = ((max(e_w, e_c) + grain - 1) // grain) * grain

    def pad_edges(ei, e):
        s = jnp.concatenate([ei[0], jnp.zeros((e_pad - e,), jnp.int32)])
        d = jnp.concatenate([ei[1], jnp.full((e_pad - e,), n_p, jnp.int32)])
        return s, d

    s_w, d_w = pad_edges(ei_writes, e_w)
    s_c, d_c = pad_edges(ei_cites, e_c)

    sc_out = _edge_phase_sc(q_tab, kp_tab, vp_tab, ka_tab, va_tab,
                            s_w, d_w, s_c, d_c, n_p, n_a, e_pad)

    return _post(sc_out, xo, W_ao, c0)


# speedup vs baseline: 31.0471x; 1.2119x over previous
import jax
import jax.numpy as jnp
from jax import lax
from jax.experimental import pallas as pl
from jax.experimental.pallas import tpu as pltpu
from jax.experimental.pallas import tpu_sc as plsc

H = 8
D = 128
DH = D // H
OUT = 16

N_TEC = 16
N_SC = 2
CHUNK = 128


def _block_diag(rel):
    eye = jnp.eye(H, dtype=rel.dtype)
    return jnp.einsum('hg,hij->higj', eye, rel).reshape(D, D)


def _proj_body(x_ref, w_ref, b_ref, o_ref):
    o_ref[...] = jnp.dot(x_ref[...], w_ref[...],
                         preferred_element_type=jnp.float32) + b_ref[...]


def _proj(x, W, b, bn=1000):
    n, d = x.shape
    c = W.shape[1]
    return pl.pallas_call(
        _proj_body,
        grid=(n // bn,),
        in_specs=[
            pl.BlockSpec((bn, d), lambda i: (i, 0)),
            pl.BlockSpec((d, c), lambda i: (0, 0)),
            pl.BlockSpec((1, c), lambda i: (0, 0)),
        ],
        out_specs=pl.BlockSpec((bn, c), lambda i: (i, 0)),
        out_shape=jax.ShapeDtypeStruct((n, c), jnp.float32),
    )(x, W, b.reshape(1, c))


def _edge_phase_sc(q_tab, kp_tab, vp_tab, ka_tab, va_tab,
                   s_w, d_w, s_c, d_c, n_p, n_a, e_pad):
    per_tec = e_pad // N_TEC
    n_chunks = per_tec // CHUNK
    DRN = 112
    acc_rows = ((n_p + N_TEC * DRN - 1) // (N_TEC * DRN)) * (N_TEC * DRN)
    rows_per_tec = acc_rows // N_TEC
    n_drain = rows_per_tec // DRN

    mesh = plsc.VectorSubcoreMesh(core_axis_name="c", subcore_axis_name="s")

    def body(q_hbm, kp_hbm, vp_hbm, ka_hbm, va_hbm,
             sw_hbm, dw_hbm, sc_hbm, dc_hbm, out_hbm,
             dbuf, sbuf, qidx_a, sidx_a, qb_a, kb_a, vb_a,
             qidx_b, sidx_b, qb_b, kb_b, vb_b, mb, zbuf, accum,
             sem, sem2, sem3, sem4, sem5, sem6):
        cid = lax.axis_index("c")
        sid = lax.axis_index("s")
        iota16 = lax.iota(jnp.int32, 16)
        e0f = jnp.where(iota16 == 0, 1.0, 0.0)

        z16 = jnp.zeros((16,), jnp.float32)

        def zb(i, _):
            zbuf[i, pl.ds(0, 16)] = z16
            zbuf[i, pl.ds(16, 16)] = z16
            return _
        lax.fori_loop(0, zbuf.shape[0], zb, None)

        for i in range(8):
            t = i // 4
            h = 2 * (i % 4) + cid
            u = t * 8 + h
            if t == 0:
                s_hbm, d_hbm, k_hbm, v_hbm, n_src = (
                    sw_hbm, dw_hbm, ka_hbm, va_hbm, n_a)
            else:
                s_hbm, d_hbm, k_hbm, v_hbm, n_src = (
                    sc_hbm, dc_hbm, kp_hbm, vp_hbm, n_p)

            def zcp(z, _):
                r0 = sid * rows_per_tec + z * DRN
                pltpu.sync_copy(zbuf, accum.at[pl.ds(r0, DRN)])
                return _
            lax.fori_loop(0, n_drain, zcp, None)
            plsc.subcore_barrier()

            qoff = h * n_p
            koff = h * n_src
            bufs = ((qidx_a, sidx_a, qb_a, kb_a, vb_a, 0, sem, sem2, sem3),
                    (qidx_b, sidx_b, qb_b, kb_b, vb_b, 1, sem4, sem5, sem6))

            def load_issue(c_idx, par):
                qidxr, sidxr, qbr, kbr, vbr, drow, s1, s2, s3 = bufs[par]
                off = sid * per_tec + c_idx * CHUNK
                pltpu.sync_copy(d_hbm.at[pl.ds(off, CHUNK)], dbuf.at[drow])
                pltpu.sync_copy(s_hbm.at[pl.ds(off, CHUNK)], sbuf)
                for j in range(CHUNK // 16):
                    dv = dbuf[drow, pl.ds(j * 16, 16)]
                    qidxr[pl.ds(j * 16, 16)] = (
                        jnp.minimum(dv, n_p - 1) + qoff)
                    sidxr[pl.ds(j * 16, 16)] = sbuf[pl.ds(j * 16, 16)] + koff
                pltpu.async_copy(q_hbm.at[qidxr], qbr, s1)
                pltpu.async_copy(k_hbm.at[sidxr], kbr, s2)
                pltpu.async_copy(v_hbm.at[sidxr], vbr, s3)

            def wait_gathers(par):
                qidxr, sidxr, qbr, kbr, vbr, drow, s1, s2, s3 = bufs[par]
                pltpu.make_async_copy(q_hbm.at[qidxr], qbr, s1).wait()
                pltpu.make_async_copy(k_hbm.at[sidxr], kbr, s2).wait()
                pltpu.make_async_copy(v_hbm.at[sidxr], vbr, s3).wait()

            def compute_scatter(par):
                qidxr, sidxr, qbr, kbr, vbr, drow, s1, s2, s3 = bufs[par]

                def group(g, carry):
                    e0 = g * 16
                    rows = e0 + iota16
                    acc = jnp.zeros((16,), jnp.float32)
                    for dh in range(DH):
                        col = jnp.full((16,), dh, jnp.int32)
                        acc = acc + (plsc.load_gather(qbr, [rows, col])
                                     * plsc.load_gather(kbr, [rows, col]))
                    wv = jnp.exp(acc)
                    for e in range(16):
                        ws = wv[e]
                        mb[e0 + e, pl.ds(0, 16)] = vbr[e0 + e] * ws
                        mb[e0 + e, pl.ds(16, 16)] = ws * e0f
                    return carry
                lax.fori_loop(0, CHUNK // 16, group, None)
                pltpu.sync_copy(mb, accum.at[dbuf.at[drow]], add=True)

            load_issue(jnp.int32(0), 0)

            def pair_body(p, _):
                for par in (0, 1):
                    c = p * 2 + par
                    load_issue(jnp.minimum(c + 1, n_chunks - 1), 1 - par)
                    wait_gathers(par)
                    compute_scatter(par)
                return _
            lax.fori_loop(0, n_chunks // 2, pair_body, None)
            wait_gathers(0)
            plsc.subcore_barrier()

            def drain(dr, _):
                r0 = sid * rows_per_tec + dr * DRN
                pltpu.sync_copy(accum.at[pl.ds(r0, DRN)],
                                mb.at[pl.ds(0, DRN)])
                pltpu.sync_copy(mb.at[pl.ds(0, DRN)],
                                out_hbm.at[u, pl.ds(r0, DRN)])
                return _
            lax.fori_loop(0, n_drain, drain, None)
            plsc.subcore_barrier()

    fn = pl.kernel(
        body,
        out_type=jax.ShapeDtypeStruct((16, acc_rows, 32), jnp.float32),
        mesh=mesh,
        compiler_params=pltpu.CompilerParams(needs_layout_passes=False,
                                             use_tc_tiling_on_sc=False),
        scratch_types=[
            pltpu.VMEM((2, CHUNK), jnp.int32),
            pltpu.VMEM((CHUNK,), jnp.int32),
            pltpu.VMEM((CHUNK,), jnp.int32),
            pltpu.VMEM((CHUNK,), jnp.int32),
            pltpu.VMEM((CHUNK, 16), jnp.float32),
            pltpu.VMEM((CHUNK, 16), jnp.float32),
            pltpu.VMEM((CHUNK, 16), jnp.float32),
            pltpu.VMEM((CHUNK,), jnp.int32),
            pltpu.VMEM((CHUNK,), jnp.int32),
            pltpu.VMEM((CHUNK, 16), jnp.float32),
            pltpu.VMEM((CHUNK, 16), jnp.float32),
            pltpu.VMEM((CHUNK, 16), jnp.float32),
            pltpu.VMEM((CHUNK, 32), jnp.float32),
            pltpu.VMEM((DRN, 32), jnp.float32),
            pltpu.VMEM_SHARED((acc_rows, 32), jnp.float32),
            pltpu.SemaphoreType.DMA,
            pltpu.SemaphoreType.DMA,
            pltpu.SemaphoreType.DMA,
            pltpu.SemaphoreType.DMA,
            pltpu.SemaphoreType.DMA,
            pltpu.SemaphoreType.DMA,
        ],
    )
    return fn(q_tab, kp_tab, vp_tab, ka_tab, va_tab, s_w, d_w, s_c, d_c)


def _post_body(sc_ref, xo_ref, wao_ref, c0_ref, o_ref):
    o = xo_ref[...] + c0_ref[...]
    for h in range(H):
        aggr_h = (sc_ref[h, :, 0:16] / (sc_ref[h, :, 16:17] + 1e-16)
                  + sc_ref[8 + h, :, 0:16] / (sc_ref[8 + h, :, 16:17] + 1e-16))
        g = jax.nn.gelu(aggr_h)
        o = o + jnp.dot(g, wao_ref[pl.ds(h * DH, DH), :],
                        preferred_element_type=jnp.float32)
    o_ref[...] = o


def _post(sc_out, xo, W_ao, c0, bn=1000):
    n = xo.shape[0]
    return pl.pallas_call(
        _post_body,
        grid=(n // bn,),
        in_specs=[
            pl.BlockSpec((16, bn, 32), lambda i: (0, i, 0)),
            pl.BlockSpec((bn, OUT), lambda i: (i, 0)),
            pl.BlockSpec((D, OUT), lambda i: (0, 0)),
            pl.BlockSpec((1, OUT), lambda i: (0, 0)),
        ],
        out_specs=pl.BlockSpec((bn, OUT), lambda i: (i, 0)),
        out_shape=jax.ShapeDtypeStruct((n, OUT), jnp.float32),
    )(sc_out, xo, W_ao, c0.reshape(1, OUT))


def kernel(x_paper, x_author, ei_writes, ei_cites, W_pre_paper, b_pre_paper,
           W_k_paper, b_k_paper, W_q_paper, b_q_paper, W_v_paper, b_v_paper,
           W_a_paper, b_a_paper, skip_paper, W_pre_author, b_pre_author,
           W_k_author, b_k_author, W_q_author, b_q_author, W_v_author,
           b_v_author, W_a_author, b_a_author, skip_author, a_rel_writes,
           m_rel_writes, p_rel_writes, a_rel_cites, m_rel_cites, p_rel_cites,
           W_out, b_out):
    n_p = x_paper.shape[0]
    n_a = x_author.shape[0]
    e_w = ei_writes.shape[1]
    e_c = ei_cites.shape[1]

    inv_sqrt = 1.0 / jnp.sqrt(jnp.float32(DH))
    A_c = _block_diag(a_rel_cites) * jnp.repeat(p_rel_cites * inv_sqrt, DH)[None, :]
    M_c = _block_diag(m_rel_cites)
    A_w = _block_diag(a_rel_writes) * jnp.repeat(p_rel_writes * inv_sqrt, DH)[None, :]
    M_w = _block_diag(m_rel_writes)

    beta = jax.nn.sigmoid(skip_paper)

    Wq_p = W_pre_paper @ W_q_paper
    bq_p = b_pre_paper @ W_q_paper + b_q_paper
    Wk_p = W_pre_paper @ (W_k_paper @ A_c)
    bk_p = (b_pre_paper @ W_k_paper + b_k_paper) @ A_c
    Wv_p = W_pre_paper @ (W_v_paper @ M_c)
    bv_p = (b_pre_paper @ W_v_paper + b_v_paper) @ M_c
    Wk_a = W_pre_author @ (W_k_author @ A_w)
    bk_a = (b_pre_author @ W_k_author + b_k_author) @ A_w
    Wv_a = W_pre_author @ (W_v_author @ M_w)
    bv_a = (b_pre_author @ W_v_author + b_v_author) @ M_w

    W_ao = (W_a_paper @ W_out) * beta
    W_xo = (W_pre_paper @ W_out) * (1.0 - beta)
    c0 = (beta * (b_a_paper @ W_out) + (1.0 - beta) * (b_pre_paper @ W_out)
          + b_out)

    Wxo_pad = jnp.zeros((D, D), jnp.float32).at[:, :OUT].set(W_xo)
    Wcat_p = jnp.concatenate([Wq_p, Wk_p, Wv_p, Wxo_pad], axis=1)
    bcat_p = jnp.concatenate([bq_p, bk_p, bv_p, jnp.zeros((D,), jnp.float32)])
    proj_p = _proj(x_paper, Wcat_p, bcat_p)
    xo = proj_p[:, 3 * D:3 * D + OUT]

    Wcat_a = jnp.concatenate([Wk_a, Wv_a], axis=1)
    bcat_a = jnp.concatenate([bk_a, bv_a])
    proj_a = _proj(x_author, Wcat_a, bcat_a)

    def headmajor(m):
        n = m.shape[0]
        return m.reshape(n, H, DH).transpose(1, 0, 2).reshape(H * n, DH)

    q_tab = headmajor(proj_p[:, 0:D])
    kp_tab = headmajor(proj_p[:, D:2 * D])
    vp_tab = headmajor(proj_p[:, 2 * D:3 * D])
    ka_tab = headmajor(proj_a[:, 0:D])
    va_tab = headmajor(proj_a[:, D:2 * D])

    grain = N_TEC * CHUNK * 2
    e_pad = ((max(e_w, e_c) + grain - 1) // grain) * grain

    def pad_edges(ei, e):
        s = jnp.concatenate([ei[0], jnp.zeros((e_pad - e,), jnp.int32)])
        d = jnp.concatenate([ei[1], jnp.full((e_pad - e,), n_p, jnp.int32)])
        return s, d

    s_w, d_w = pad_edges(ei_writes, e_w)
    s_c, d_c = pad_edges(ei_cites, e_c)

    sc_out = _edge_phase_sc(q_tab, kp_tab, vp_tab, ka_tab, va_tab,
                            s_w, d_w, s_c, d_c, n_p, n_a, e_pad)

    return _post(sc_out, xo, W_ao, c0)
